# baseline (speedup 1.0000x reference)
import functools

import jax
import jax.numpy as jnp
from jax import lax
from jax.experimental import pallas as pl
from jax.experimental.pallas import tpu as pltpu
from jax.experimental.pallas import tpu_sc as plsc

D_MODEL = 64
D_PAD = 128
SEQ = 200
BATCH = 4096
BT = BATCH // 128
ITEMS = SEQ * BT
NUM_CORES = 2
NUM_SUBCORES = 16
NUM_WORKERS = NUM_CORES * NUM_SUBCORES
ITEMS_PER_W = ITEMS // NUM_WORKERS


def _gather_embed(wp, ids_t):
    mesh = plsc.VectorSubcoreMesh(core_axis_name="c", subcore_axis_name="s")

    @functools.partial(
        pl.kernel,
        mesh=mesh,
        out_type=jax.ShapeDtypeStruct((SEQ, D_MODEL, BATCH), jnp.float32),
        scratch_types=[
            pltpu.VMEM((8, 128), jnp.int32),
            pltpu.VMEM((128, D_PAD), jnp.float32),
            pltpu.VMEM((D_MODEL, 128), jnp.float32),
            pltpu.SemaphoreType.DMA,
        ],
        compiler_params=pltpu.CompilerParams(use_tc_tiling_on_sc=True,
                                             needs_layout_passes=False),
    )
    def k(table_hbm, idx_hbm, out_hbm, idx_v, g_v, t_v, sem):
        wid = lax.axis_index("s") * NUM_CORES + lax.axis_index("c")
        item0 = wid * ITEMS_PER_W
        lane = lax.iota(jnp.int32, 16)

        def body(i, carry):
            item = item0 + i
            s = item // BT
            bt = item % BT
            pltpu.sync_copy(idx_hbm.at[s, pl.ds(bt * 128, 128)],
                            idx_v.at[0])
            pltpu.async_copy(table_hbm.at[idx_v.at[0]], g_v, sem).wait()

            def trans(d, c):
                for t0 in range(8):
                    vals = plsc.load_gather(
                        g_v, [lane + (t0 * 16), jnp.full((16,), d, jnp.int32)])
                    t_v[d, pl.ds(t0 * 16, 16)] = vals
                return c

            lax.fori_loop(0, D_MODEL, trans, 0)
            pltpu.sync_copy(t_v, out_hbm.at[s, :, pl.ds(bt * 128, 128)])
            return carry

        lax.fori_loop(0, ITEMS_PER_W, body, 0)

    return k(wp, ids_t)


def kernel(input_ids, weight):
    wp = jnp.pad(weight, ((0, 0), (0, D_PAD - D_MODEL)))
    ids_t = input_ids.T.astype(jnp.int32)
    out = _gather_embed(wp, ids_t)
    return out.transpose(2, 0, 1)


# speedup vs baseline: 1.1423x; 1.1423x over previous
import functools

import jax
import jax.numpy as jnp
from jax import lax
from jax.experimental import pallas as pl
from jax.experimental.pallas import tpu as pltpu
from jax.experimental.pallas import tpu_sc as plsc

D_MODEL = 64
D_PAD = 128
SEQ = 200
BATCH = 4096
TOK_BLK = 256
BLKS = BATCH // TOK_BLK
ITEMS = SEQ * BLKS
NUM_CORES = 2
NUM_SUBCORES = 16
NUM_WORKERS = NUM_CORES * NUM_SUBCORES
ITEMS_PER_W = ITEMS // NUM_WORKERS


def _gather_embed(wp, ids_t):
    mesh = plsc.VectorSubcoreMesh(core_axis_name="c", subcore_axis_name="s")

    @functools.partial(
        pl.kernel,
        mesh=mesh,
        out_type=jax.ShapeDtypeStruct((SEQ, D_MODEL, BATCH), jnp.float32),
        scratch_types=[
            pltpu.VMEM((2, 2, 128), jnp.int32),
            pltpu.VMEM((2, TOK_BLK, D_PAD), jnp.float32),
            pltpu.VMEM((D_MODEL, TOK_BLK), jnp.float32),
            [pltpu.SemaphoreType.DMA] * 2,
        ],
        compiler_params=pltpu.CompilerParams(use_tc_tiling_on_sc=True,
                                             needs_layout_passes=False),
    )
    def k(table_hbm, idx_hbm, out_hbm, idx_v, g_v, t_v, sems):
        wid = lax.axis_index("s") * NUM_CORES + lax.axis_index("c")
        item0 = wid * ITEMS_PER_W
        lane = lax.iota(jnp.int32, 16)

        def stage(i, b):
            item = item0 + i
            s = item // BLKS
            b0 = (item % BLKS) * TOK_BLK
            for j in range(2):
                pltpu.sync_copy(idx_hbm.at[s, pl.ds(b0 + j * 128, 128)],
                                idx_v.at[b, j])
            for j in range(2):
                pltpu.async_copy(table_hbm.at[idx_v.at[b, j]],
                                 g_v.at[b, pl.ds(j * 128, 128)],
                                 sems[b])

        def drain(b):
            for j in range(2):
                pltpu.make_async_copy(
                    table_hbm.at[idx_v.at[b, j]],
                    g_v.at[b, pl.ds(j * 128, 128)],
                    sems[b],
                ).wait()

        def flush(i, b):
            item = item0 + i
            s = item // BLKS
            b0 = (item % BLKS) * TOK_BLK
            src = g_v.at[b]

            def trans(d, c):
                for tg in range(TOK_BLK // 16):
                    vals = plsc.load_gather(
                        src,
                        [lane + (tg * 16), jnp.full((16,), d, jnp.int32)])
                    t_v[d, pl.ds(tg * 16, 16)] = vals
                return c

            lax.fori_loop(0, D_MODEL, trans, 0)
            pltpu.sync_copy(t_v, out_hbm.at[s, :, pl.ds(b0, TOK_BLK)])

        stage(0, 0)

        def body2(g, carry):
            i = g * 2
            drain(0)
            stage(i + 1, 1)
            flush(i, 0)
            drain(1)
            stage(i + 2, 0)
            flush(i + 1, 1)
            return carry

        lax.fori_loop(0, (ITEMS_PER_W - 2) // 2, body2, 0)
        i_last = ITEMS_PER_W - 2
        drain(0)
        stage(i_last + 1, 1)
        flush(i_last, 0)
        drain(1)
        flush(i_last + 1, 1)

    return k(wp, ids_t)


def kernel(input_ids, weight):
    wp = jnp.pad(weight, ((0, 0), (0, D_PAD - D_MODEL)))
    ids_t = input_ids.T.astype(jnp.int32)
    out = _gather_embed(wp, ids_t)
    return out.transpose(2, 0, 1)


# speedup vs baseline: 1.1427x; 1.0003x over previous
import functools

import jax
import jax.numpy as jnp
from jax import lax
from jax.experimental import pallas as pl
from jax.experimental.pallas import tpu as pltpu
from jax.experimental.pallas import tpu_sc as plsc

D_MODEL = 64
D_PAD = 128
SEQ = 200
BATCH = 4096
TOK_BLK = 256
BLKS = BATCH // TOK_BLK
ITEMS = SEQ * BLKS
NUM_CORES = 2
NUM_SUBCORES = 16
NUM_WORKERS = NUM_CORES * NUM_SUBCORES
ITEMS_PER_W = ITEMS // NUM_WORKERS


def _gather_embed(wp, ids_t):
    mesh = plsc.VectorSubcoreMesh(core_axis_name="c", subcore_axis_name="s")

    @functools.partial(
        pl.kernel,
        mesh=mesh,
        out_type=jax.ShapeDtypeStruct((SEQ, D_MODEL, BATCH), jnp.float32),
        scratch_types=[
            pltpu.VMEM((2, 2, 128), jnp.int32),
            pltpu.VMEM((2, TOK_BLK, D_PAD), jnp.float32),
            pltpu.VMEM((D_MODEL, TOK_BLK), jnp.float32),
            [pltpu.SemaphoreType.DMA] * 2,
        ],
        compiler_params=pltpu.CompilerParams(use_tc_tiling_on_sc=True,
                                             needs_layout_passes=False),
    )
    def k(table_hbm, idx_hbm, out_hbm, idx_v, g_v, t_v, sems):
        wid = lax.axis_index("s") * NUM_CORES + lax.axis_index("c")
        item0 = wid * ITEMS_PER_W
        lane = lax.iota(jnp.int32, 16)

        def stage(i, b):
            item = item0 + i
            s = item // BLKS
            b0 = (item % BLKS) * TOK_BLK
            with jax.named_scope("idx_copy"):
                for j in range(2):
                    pltpu.sync_copy(idx_hbm.at[s, pl.ds(b0 + j * 128, 128)],
                                    idx_v.at[b, j])
            with jax.named_scope("fire_gather"):
                for j in range(2):
                    pltpu.async_copy(table_hbm.at[idx_v.at[b, j]],
                                     g_v.at[b, pl.ds(j * 128, 128)],
                                     sems[b])

        def drain(b):
            with jax.named_scope("drain_gather"):
                for j in range(2):
                    pltpu.make_async_copy(
                        table_hbm.at[idx_v.at[b, j]],
                        g_v.at[b, pl.ds(j * 128, 128)],
                        sems[b],
                    ).wait()

        def flush(i, b):
            item = item0 + i
            s = item // BLKS
            b0 = (item % BLKS) * TOK_BLK
            src = g_v.at[b]

            def trans(d, c):
                for tg in range(TOK_BLK // 16):
                    vals = plsc.load_gather(
                        src,
                        [lane + (tg * 16), jnp.full((16,), d, jnp.int32)])
                    t_v[d, pl.ds(tg * 16, 16)] = vals
                return c

            with jax.named_scope("transpose"):
                lax.fori_loop(0, D_MODEL, trans, 0)
            with jax.named_scope("store_out"):
                pltpu.sync_copy(t_v, out_hbm.at[s, :, pl.ds(b0, TOK_BLK)])

        stage(0, 0)

        def body2(g, carry):
            i = g * 2
            drain(0)
            stage(i + 1, 1)
            flush(i, 0)
            drain(1)
            stage(i + 2, 0)
            flush(i + 1, 1)
            return carry

        lax.fori_loop(0, (ITEMS_PER_W - 2) // 2, body2, 0)
        i_last = ITEMS_PER_W - 2
        drain(0)
        stage(i_last + 1, 1)
        flush(i_last, 0)
        drain(1)
        flush(i_last + 1, 1)

    return k(wp, ids_t)


def kernel(input_ids, weight):
    wp = jnp.pad(weight, ((0, 0), (0, D_PAD - D_MODEL)))
    ids_t = input_ids.T.astype(jnp.int32)
    out = _gather_embed(wp, ids_t)
    return out.transpose(2, 0, 1)


# speedup vs baseline: 1.7036x; 1.4909x over previous
import functools

import jax
import jax.numpy as jnp
from jax import lax
from jax.experimental import pallas as pl
from jax.experimental.pallas import tpu as pltpu
from jax.experimental.pallas import tpu_sc as plsc

D_MODEL = 64
D_PAD = 128
SEQ = 200
BATCH = 4096
TOK_BLK = 256
BLKS = BATCH // TOK_BLK
ITEMS = SEQ * BLKS
NUM_CORES = 2
NUM_SUBCORES = 16
NUM_WORKERS = NUM_CORES * NUM_SUBCORES
ITEMS_PER_W = ITEMS // NUM_WORKERS


def _gather_embed(wp, ids_t):
    mesh = plsc.VectorSubcoreMesh(core_axis_name="c", subcore_axis_name="s")

    @functools.partial(
        pl.kernel,
        mesh=mesh,
        out_type=jax.ShapeDtypeStruct((SEQ, D_MODEL, BATCH), jnp.float32),
        scratch_types=[
            pltpu.VMEM((2, 2, 128), jnp.int32),
            pltpu.VMEM((2, TOK_BLK, D_PAD), jnp.float32),
            pltpu.VMEM((D_MODEL, TOK_BLK), jnp.float32),
            [pltpu.SemaphoreType.DMA] * 2,
        ],
        compiler_params=pltpu.CompilerParams(use_tc_tiling_on_sc=True,
                                             needs_layout_passes=False),
    )
    def k(table_hbm, idx_hbm, out_hbm, idx_v, g_v, t_v, sems):
        wid = lax.axis_index("s") * NUM_CORES + lax.axis_index("c")
        item0 = wid * ITEMS_PER_W
        lane = lax.iota(jnp.int32, 16)
        rot = [lax.rem(lane + r, 16) for r in range(16)]

        def stage(i, b):
            item = item0 + i
            s = item // BLKS
            b0 = (item % BLKS) * TOK_BLK
            with jax.named_scope("idx_copy"):
                for j in range(2):
                    pltpu.sync_copy(idx_hbm.at[s, pl.ds(b0 + j * 128, 128)],
                                    idx_v.at[b, j])
            with jax.named_scope("fire_gather"):
                for j in range(2):
                    pltpu.async_copy(table_hbm.at[idx_v.at[b, j]],
                                     g_v.at[b, pl.ds(j * 128, 128)],
                                     sems[b])

        def drain(b):
            with jax.named_scope("drain_gather"):
                for j in range(2):
                    pltpu.make_async_copy(
                        table_hbm.at[idx_v.at[b, j]],
                        g_v.at[b, pl.ds(j * 128, 128)],
                        sems[b],
                    ).wait()

        def flush(i, b):
            item = item0 + i
            s = item // BLKS
            b0 = (item % BLKS) * TOK_BLK
            src = g_v.at[b]

            def trans(tg, c):
                row = lane + tg * 16
                for dg in range(D_MODEL // 16):
                    d0 = dg * 16
                    for r in range(16):
                        col = rot[r] + d0
                        vals = plsc.load_gather(src, [row, col])
                        plsc.store_scatter(t_v, [col, row], vals)
                return c

            with jax.named_scope("transpose"):
                lax.fori_loop(0, TOK_BLK // 16, trans, 0)
            with jax.named_scope("store_out"):
                pltpu.sync_copy(t_v, out_hbm.at[s, :, pl.ds(b0, TOK_BLK)])

        stage(0, 0)

        def body2(g, carry):
            i = g * 2
            drain(0)
            stage(i + 1, 1)
            flush(i, 0)
            drain(1)
            stage(i + 2, 0)
            flush(i + 1, 1)
            return carry

        lax.fori_loop(0, (ITEMS_PER_W - 2) // 2, body2, 0)
        i_last = ITEMS_PER_W - 2
        drain(0)
        stage(i_last + 1, 1)
        flush(i_last, 0)
        drain(1)
        flush(i_last + 1, 1)

    return k(wp, ids_t)


def kernel(input_ids, weight):
    wp = jnp.pad(weight, ((0, 0), (0, D_PAD - D_MODEL)))
    ids_t = input_ids.T.astype(jnp.int32)
    out = _gather_embed(wp, ids_t)
    return out.transpose(2, 0, 1)


# speedup vs baseline: 2.1910x; 1.2861x over previous
import functools

import jax
import jax.numpy as jnp
from jax import lax
from jax.experimental import pallas as pl
from jax.experimental.pallas import tpu as pltpu
from jax.experimental.pallas import tpu_sc as plsc

D_MODEL = 64
D_PAD = 128
SEQ = 200
BATCH = 4096
TOK_BLK = 256
BLKS = BATCH // TOK_BLK
ITEMS = SEQ * BLKS
NUM_CORES = 2
NUM_SUBCORES = 16
NUM_WORKERS = NUM_CORES * NUM_SUBCORES
ITEMS_PER_W = ITEMS // NUM_WORKERS


def _gather_embed(wp, ids_t):
    mesh = plsc.VectorSubcoreMesh(core_axis_name="c", subcore_axis_name="s")

    @functools.partial(
        pl.kernel,
        mesh=mesh,
        out_type=jax.ShapeDtypeStruct((SEQ, D_MODEL, BATCH), jnp.float32),
        scratch_types=[
            pltpu.VMEM((2, 2, 128), jnp.int32),
            pltpu.VMEM((2, TOK_BLK, D_PAD), jnp.float32),
            pltpu.VMEM((D_MODEL, TOK_BLK), jnp.float32),
            [pltpu.SemaphoreType.DMA] * 2,
        ],
        compiler_params=pltpu.CompilerParams(use_tc_tiling_on_sc=True,
                                             needs_layout_passes=False),
    )
    def k(table_hbm, idx_hbm, out_hbm, idx_v, g_v, t_v, sems):
        wid = lax.axis_index("s") * NUM_CORES + lax.axis_index("c")
        item0 = wid * ITEMS_PER_W
        lane = lax.iota(jnp.int32, 16)
        rot = [lax.rem(lane + r, 16) for r in range(16)]

        def stage(i, b):
            item = item0 + i
            s = item // BLKS
            b0 = (item % BLKS) * TOK_BLK
            with jax.named_scope("idx_copy"):
                for j in range(2):
                    pltpu.sync_copy(idx_hbm.at[s, pl.ds(b0 + j * 128, 128)],
                                    idx_v.at[b, j])
            with jax.named_scope("fire_gather"):
                for j in range(2):
                    pltpu.async_copy(table_hbm.at[idx_v.at[b, j]],
                                     g_v.at[b, pl.ds(j * 128, 128)],
                                     sems[b])

        def drain(b):
            with jax.named_scope("drain_gather"):
                for j in range(2):
                    pltpu.make_async_copy(
                        table_hbm.at[idx_v.at[b, j]],
                        g_v.at[b, pl.ds(j * 128, 128)],
                        sems[b],
                    ).wait()

        def flush(i, b):
            item = item0 + i
            s = item // BLKS
            b0 = (item % BLKS) * TOK_BLK
            src = g_v.at[b]

            with jax.named_scope("transpose"):
                @plsc.parallel_loop(0, TOK_BLK // 16, unroll=2)
                def trans(tg):
                    row = lane + tg * 16
                    for dg in range(D_MODEL // 16):
                        d0 = dg * 16
                        for r in range(16):
                            col = rot[r] + d0
                            vals = plsc.load_gather(src, [row, col])
                            plsc.store_scatter(t_v, [col, row], vals)
            with jax.named_scope("store_out"):
                pltpu.sync_copy(t_v, out_hbm.at[s, :, pl.ds(b0, TOK_BLK)])

        stage(0, 0)

        def body2(g, carry):
            i = g * 2
            drain(0)
            stage(i + 1, 1)
            flush(i, 0)
            drain(1)
            stage(i + 2, 0)
            flush(i + 1, 1)
            return carry

        lax.fori_loop(0, (ITEMS_PER_W - 2) // 2, body2, 0)
        i_last = ITEMS_PER_W - 2
        drain(0)
        stage(i_last + 1, 1)
        flush(i_last, 0)
        drain(1)
        flush(i_last + 1, 1)

    return k(wp, ids_t)


def kernel(input_ids, weight):
    wp = jnp.pad(weight, ((0, 0), (0, D_PAD - D_MODEL)))
    ids_t = input_ids.T.astype(jnp.int32)
    out = _gather_embed(wp, ids_t)
    return out.transpose(2, 0, 1)
